# baseline (speedup 1.0000x reference)
import functools

import jax
import jax.numpy as jnp
import numpy as np
from jax import lax
from jax.experimental import pallas as pl
from jax.experimental.pallas import tpu as pltpu
from jax.experimental.pallas import tpu_sc as plsc

B, S, D, H, F, V = 2, 2048, 768, 12, 3072, 100000
DH = D // H
N = B * S


_NW = 32
_BPW = N // _NW


def _sc_gather(table, idx):
  mesh = plsc.VectorSubcoreMesh(core_axis_name="c", subcore_axis_name="s")

  @functools.partial(
      pl.kernel,
      mesh=mesh,
      out_type=jax.ShapeDtypeStruct((N, D), jnp.float32),
      scratch_types=[
          pltpu.VMEM((_BPW,), jnp.int32),
          pltpu.VMEM((_BPW, D), jnp.float32),
          pltpu.SemaphoreType.DMA,
      ],
  )
  def k(table_hbm, idx_hbm, out_hbm, idx_v, rows_v, sem):
    wid = lax.axis_index("s") * 2 + lax.axis_index("c")
    base = wid * _BPW
    pltpu.sync_copy(idx_hbm.at[pl.ds(base, _BPW)], idx_v)
    pltpu.async_copy(table_hbm.at[idx_v], rows_v, sem).wait()
    pltpu.sync_copy(rows_v, out_hbm.at[pl.ds(base, _BPW)])

  return k(table, idx)


_BM = 512


def _qkv_body(x_ref, w_ref, b_ref, o_ref):
  o_ref[...] = (
      jnp.dot(x_ref[...], w_ref[...], preferred_element_type=jnp.float32)
      + b_ref[...]
  )


def _qkv(x, wqkv, bqkv):
  return pl.pallas_call(
      _qkv_body,
      grid=(N // _BM,),
      in_specs=[
          pl.BlockSpec((_BM, D), lambda m: (m, 0)),
          pl.BlockSpec((D, 3 * D), lambda m: (0, 0)),
          pl.BlockSpec((1, 3 * D), lambda m: (0, 0)),
      ],
      out_specs=pl.BlockSpec((_BM, 3 * D), lambda m: (m, 0)),
      out_shape=jax.ShapeDtypeStruct((N, 3 * D), jnp.float32),
  )(x, wqkv, bqkv)


def _attn_body(q_ref, k_ref, v_ref, m_ref, o_ref):
  q = q_ref[0, 0]
  k = k_ref[0, 0]
  v = v_ref[0, 0]
  s = lax.dot_general(
      q, k, (((1,), (1,)), ((), ())), preferred_element_type=jnp.float32
  )
  s = s * (1.0 / np.sqrt(DH)) + m_ref[...]
  p = jax.nn.softmax(s, axis=-1)
  o_ref[0, 0] = jnp.dot(p, v, preferred_element_type=jnp.float32)


def _attention(q, k, v, mask):
  qkv_spec = pl.BlockSpec((1, 1, S, DH), lambda b, h: (b, h, 0, 0))
  return pl.pallas_call(
      _attn_body,
      grid=(B, H),
      in_specs=[
          qkv_spec,
          qkv_spec,
          qkv_spec,
          pl.BlockSpec((S, S), lambda b, h: (0, 0)),
      ],
      out_specs=qkv_spec,
      out_shape=jax.ShapeDtypeStruct((B, H, S, DH), jnp.float32),
  )(q, k, v, mask)


def _ln(r, g, b):
  m = r.mean(-1, keepdims=True)
  v = ((r - m) ** 2).mean(-1, keepdims=True)
  return (r - m) / jnp.sqrt(v + 1e-5) * g + b


def _oproj_body(o_ref, x_ref, w_ref, b_ref, g_ref, be_ref, out_ref):
  o2 = (
      jnp.dot(o_ref[...], w_ref[...], preferred_element_type=jnp.float32)
      + b_ref[...]
  )
  out_ref[...] = _ln(x_ref[...] + o2, g_ref[...], be_ref[...])


def _oproj_ln(o, x, wo, bo, g1, be1):
  return pl.pallas_call(
      _oproj_body,
      grid=(N // _BM,),
      in_specs=[
          pl.BlockSpec((_BM, D), lambda m: (m, 0)),
          pl.BlockSpec((_BM, D), lambda m: (m, 0)),
          pl.BlockSpec((D, D), lambda m: (0, 0)),
          pl.BlockSpec((1, D), lambda m: (0, 0)),
          pl.BlockSpec((1, D), lambda m: (0, 0)),
          pl.BlockSpec((1, D), lambda m: (0, 0)),
      ],
      out_specs=pl.BlockSpec((_BM, D), lambda m: (m, 0)),
      out_shape=jax.ShapeDtypeStruct((N, D), jnp.float32),
  )(o, x, wo, bo, g1, be1)


def _ffn_body(x_ref, w1_ref, b1_ref, w2_ref, b2_ref, g_ref, be_ref, out_ref):
  x1 = x_ref[...]
  h = jnp.maximum(
      jnp.dot(x1, w1_ref[...], preferred_element_type=jnp.float32)
      + b1_ref[...],
      0.0,
  )
  y = (
      jnp.dot(h, w2_ref[...], preferred_element_type=jnp.float32)
      + b2_ref[...]
  )
  out_ref[...] = _ln(x1 + y, g_ref[...], be_ref[...])


def _ffn_ln(x1, w1, b1, w2, b2, g2, be2):
  return pl.pallas_call(
      _ffn_body,
      grid=(N // _BM,),
      in_specs=[
          pl.BlockSpec((_BM, D), lambda m: (m, 0)),
          pl.BlockSpec((D, F), lambda m: (0, 0)),
          pl.BlockSpec((1, F), lambda m: (0, 0)),
          pl.BlockSpec((F, D), lambda m: (0, 0)),
          pl.BlockSpec((1, D), lambda m: (0, 0)),
          pl.BlockSpec((1, D), lambda m: (0, 0)),
          pl.BlockSpec((1, D), lambda m: (0, 0)),
      ],
      out_specs=pl.BlockSpec((_BM, D), lambda m: (m, 0)),
      out_shape=jax.ShapeDtypeStruct((N, D), jnp.float32),
  )(x1, w1, b1, w2, b2, g2, be2)


def kernel(input, mask, table, Wq, bq, Wk, bk, Wv, bv, Wo, bo, W1, b1, W2, b2,
           g1, be1, g2, be2):
  idx = input.reshape(N).astype(jnp.int32)
  x = _sc_gather(table, idx)

  wqkv = jnp.concatenate([Wq, Wk, Wv], axis=1)
  bqkv = jnp.concatenate([bq, bk, bv]).reshape(1, 3 * D)
  qkv = _qkv(x, wqkv, bqkv)

  qkv_t = qkv.reshape(B, S, 3, H, DH).transpose(2, 0, 3, 1, 4)
  o_t = _attention(qkv_t[0], qkv_t[1], qkv_t[2], mask)
  o = o_t.transpose(0, 2, 1, 3).reshape(N, D)

  x1 = _oproj_ln(o, x, Wo, bo.reshape(1, D), g1.reshape(1, D),
                 be1.reshape(1, D))
  x2 = _ffn_ln(x1, W1, b1.reshape(1, F), W2, b2.reshape(1, D),
               g2.reshape(1, D), be2.reshape(1, D))
  return x2.reshape(B, S, D)


# speedup vs baseline: 1.7020x; 1.7020x over previous
import functools

import jax
import jax.numpy as jnp
import numpy as np
from jax import lax
from jax.experimental import pallas as pl
from jax.experimental.pallas import tpu as pltpu
from jax.experimental.pallas import tpu_sc as plsc

B, S, D, H, F, V = 2, 2048, 768, 12, 3072, 100000
DH = D // H
N = B * S


_NW = 32
_BPW = N // _NW


def _sc_gather(table, idx):
  mesh = plsc.VectorSubcoreMesh(core_axis_name="c", subcore_axis_name="s")

  @functools.partial(
      pl.kernel,
      mesh=mesh,
      out_type=jax.ShapeDtypeStruct((N, D), jnp.float32),
      scratch_types=[
          pltpu.VMEM((_BPW,), jnp.int32),
          pltpu.VMEM((_BPW, D), jnp.float32),
          pltpu.SemaphoreType.DMA,
      ],
  )
  def k(table_hbm, idx_hbm, out_hbm, idx_v, rows_v, sem):
    wid = lax.axis_index("s") * 2 + lax.axis_index("c")
    base = wid * _BPW
    pltpu.sync_copy(idx_hbm.at[pl.ds(base, _BPW)], idx_v)
    pltpu.async_copy(table_hbm.at[idx_v], rows_v, sem).wait()
    pltpu.sync_copy(rows_v, out_hbm.at[pl.ds(base, _BPW)])

  return k(table, idx)


_BM = 512


def _qkv_body(x_ref, wq_ref, bq_ref, wk_ref, bk_ref, wv_ref, bv_ref,
              q_ref, k_ref, v_ref):
  x = x_ref[...]
  q_ref[...] = (
      jnp.dot(x, wq_ref[...], preferred_element_type=jnp.float32) + bq_ref[...]
  )
  k_ref[...] = (
      jnp.dot(x, wk_ref[...], preferred_element_type=jnp.float32) + bk_ref[...]
  )
  v_ref[...] = (
      jnp.dot(x, wv_ref[...], preferred_element_type=jnp.float32) + bv_ref[...]
  )


def _qkv(x, wq, bq, wk, bk, wv, bv):
  row_spec = pl.BlockSpec((_BM, D), lambda m: (m, 0))
  w_spec = pl.BlockSpec((D, D), lambda m: (0, 0))
  b_spec = pl.BlockSpec((1, D), lambda m: (0, 0))
  out = jax.ShapeDtypeStruct((N, D), jnp.float32)
  return pl.pallas_call(
      _qkv_body,
      grid=(N // _BM,),
      in_specs=[row_spec, w_spec, b_spec, w_spec, b_spec, w_spec, b_spec],
      out_specs=[row_spec, row_spec, row_spec],
      out_shape=[out, out, out],
  )(x, wq, bq, wk, bk, wv, bv)


def _attn_body(q_ref, k_ref, v_ref, m_ref, o_ref):
  mask = m_ref[...]
  scale = 1.0 / np.sqrt(DH)
  outs = []
  for i in range(2):
    q = q_ref[0, :, i * DH:(i + 1) * DH]
    k = k_ref[0, :, i * DH:(i + 1) * DH]
    v = v_ref[0, :, i * DH:(i + 1) * DH]
    s = lax.dot_general(
        q, k, (((1,), (1,)), ((), ())), preferred_element_type=jnp.float32
    )
    s = s * scale + mask
    p = jax.nn.softmax(s, axis=-1)
    outs.append(jnp.dot(p, v, preferred_element_type=jnp.float32))
  o_ref[0] = jnp.concatenate(outs, axis=-1)


def _attention(q, k, v, mask):
  hp_spec = pl.BlockSpec((1, S, 2 * DH), lambda b, h: (b, 0, h))
  return pl.pallas_call(
      _attn_body,
      grid=(B, H // 2),
      in_specs=[
          hp_spec,
          hp_spec,
          hp_spec,
          pl.BlockSpec((S, S), lambda b, h: (0, 0)),
      ],
      out_specs=hp_spec,
      out_shape=jax.ShapeDtypeStruct((B, S, D), jnp.float32),
  )(q, k, v, mask)


def _ln(r, g, b):
  m = r.mean(-1, keepdims=True)
  v = ((r - m) ** 2).mean(-1, keepdims=True)
  return (r - m) / jnp.sqrt(v + 1e-5) * g + b


def _post_body(o_ref, x_ref, wo_ref, bo_ref, g1_ref, be1_ref,
               w1_ref, b1_ref, w2_ref, b2_ref, g2_ref, be2_ref, out_ref):
  o2 = (
      jnp.dot(o_ref[...], wo_ref[...], preferred_element_type=jnp.float32)
      + bo_ref[...]
  )
  x1 = _ln(x_ref[...] + o2, g1_ref[...], be1_ref[...])
  h = jnp.maximum(
      jnp.dot(x1, w1_ref[...], preferred_element_type=jnp.float32)
      + b1_ref[...],
      0.0,
  )
  y = (
      jnp.dot(h, w2_ref[...], preferred_element_type=jnp.float32)
      + b2_ref[...]
  )
  out_ref[...] = _ln(x1 + y, g2_ref[...], be2_ref[...])


def _post(o, x, wo, bo, g1, be1, w1, b1, w2, b2, g2, be2):
  row_spec = pl.BlockSpec((_BM, D), lambda m: (m, 0))
  d_spec = pl.BlockSpec((1, D), lambda m: (0, 0))
  return pl.pallas_call(
      _post_body,
      grid=(N // _BM,),
      in_specs=[
          row_spec,
          row_spec,
          pl.BlockSpec((D, D), lambda m: (0, 0)),
          d_spec,
          d_spec,
          d_spec,
          pl.BlockSpec((D, F), lambda m: (0, 0)),
          pl.BlockSpec((1, F), lambda m: (0, 0)),
          pl.BlockSpec((F, D), lambda m: (0, 0)),
          d_spec,
          d_spec,
          d_spec,
      ],
      out_specs=row_spec,
      out_shape=jax.ShapeDtypeStruct((N, D), jnp.float32),
  )(o, x, wo, bo, g1, be1, w1, b1, w2, b2, g2, be2)


def kernel(input, mask, table, Wq, bq, Wk, bk, Wv, bv, Wo, bo, W1, b1, W2, b2,
           g1, be1, g2, be2):
  idx = input.reshape(N).astype(jnp.int32)
  x = _sc_gather(table, idx)

  q, k, v = _qkv(x, Wq, bq.reshape(1, D), Wk, bk.reshape(1, D),
                 Wv, bv.reshape(1, D))
  o = _attention(q.reshape(B, S, D), k.reshape(B, S, D), v.reshape(B, S, D),
                 mask)
  x2 = _post(o.reshape(N, D), x, Wo, bo.reshape(1, D), g1.reshape(1, D),
             be1.reshape(1, D), W1, b1.reshape(1, F), W2, b2.reshape(1, D),
             g2.reshape(1, D), be2.reshape(1, D))
  return x2.reshape(B, S, D)
